# Initial kernel scaffold; baseline (speedup 1.0000x reference)
import jax
import jax.numpy as jnp
from jax.experimental import pallas as pl


def kernel(idx, emb_table):
    raise NotImplementedError("write your pallas kernel here")


# speedup vs baseline: 12.2133x; 12.2133x over previous
import functools

import jax
import jax.numpy as jnp
from jax.experimental import pallas as pl
from jax.experimental.pallas import tpu as pltpu

_TOPK = 32
_NITERS = 30


def _adj_block_kernel(wb_ref, w_ref, out_ref):
    w = w_ref[...]
    wb = wb_ref[...]
    sq = w * w
    ones = jnp.ones((1, w.shape[1]), dtype=jnp.float32)
    col_sumsq = jax.lax.dot_general(
        ones, sq, (((1,), (1,)), ((), ())),
        preferred_element_type=jnp.float32,
        precision=jax.lax.Precision.HIGHEST)
    norm_cols = jnp.sqrt(col_sumsq)
    norm_rows = jnp.sqrt(
        jnp.sum(wb * wb, axis=1, keepdims=True))
    g = jax.lax.dot_general(
        wb, w, (((1,), (1,)), ((), ())),
        preferred_element_type=jnp.float32)
    cos = g / (norm_rows * norm_cols)

    b = cos.shape[0]
    lo = jnp.full((b, 1), -1.03, dtype=jnp.float32)
    hi = jnp.full((b, 1), 1.03, dtype=jnp.float32)

    def body(_, carry):
        lo, hi = carry
        mid = 0.5 * (lo + hi)
        cnt = jnp.sum((cos >= mid).astype(jnp.float32), axis=1, keepdims=True)
        ge = cnt >= _TOPK
        return jnp.where(ge, mid, lo), jnp.where(ge, hi, mid)

    lo, _ = jax.lax.fori_loop(0, _NITERS, body, (lo, hi))
    out_ref[...] = jnp.where(cos >= lo, cos, 0.0)


@functools.partial(jax.jit, static_argnames=())
def _build_adj(weights):
    n, d = weights.shape
    block = 200
    if n % block != 0:
        block = n
    grid = n // block
    return pl.pallas_call(
        _adj_block_kernel,
        grid=(grid,),
        in_specs=[
            pl.BlockSpec((block, d), lambda i: (i, 0)),
            pl.BlockSpec((n, d), lambda i: (0, 0)),
        ],
        out_specs=pl.BlockSpec((block, n), lambda i: (i, 0)),
        out_shape=jax.ShapeDtypeStruct((n, n), jnp.float32),
        compiler_params=pltpu.CompilerParams(
            dimension_semantics=("arbitrary",),
        ),
    )(weights, weights)


def kernel(idx, emb_table):
    weights = jnp.take(emb_table, idx, axis=0).reshape(idx.shape[0], -1)
    return _build_adj(weights)
